# baseline (speedup 1.0000x reference)
import jax
import jax.numpy as jnp
from jax.experimental import pallas as pl
from jax.experimental.pallas import tpu as pltpu

_D_MODEL = 512
_D_STATE = 64
_HEADDIM = 64
_NHEADS = 16
_D_INNER = 1024
_CONV_DIM = _D_INNER + 2 * _D_STATE
_D_IN_PROJ = 2 * _D_INNER + 2 * _D_STATE + _NHEADS
_SEQLEN = 1024
_BATCH = 2
_DEPTH = 2
_CHUNK = 256
_NCHUNKS = _SEQLEN // _CHUNK
_R = _BATCH * _CHUNK
_LOG2E = 1.4426950408889634


def _stack_kernel(x_ref, w_in_ref, convw_ref, convb_ref, dtb_ref, alog_ref,
                  dbig_ref, normw_ref, w_out_ref, e64_ref, e256_ref,
                  out_ref, act_ref, carry_ref, h_ref):
    l = pl.program_id(0)
    c = pl.program_id(1)

    @pl.when(c == 0)
    def _():
        carry_ref[...] = jnp.zeros_like(carry_ref)
        h_ref[...] = jnp.zeros_like(h_ref)

    xin = x_ref[0]
    prev0 = act_ref[0, pl.ds(c * _CHUNK, _CHUNK), :]
    prev1 = act_ref[1, pl.ds(c * _CHUNK, _CHUNK), :]
    xb = jnp.where(l == 0,
                   jnp.concatenate([xin[0], xin[1]], axis=0),
                   jnp.concatenate([prev0, prev1], axis=0))

    zxbcdt = jnp.dot(xb, w_in_ref[0],
                     preferred_element_type=jnp.float32)
    z = zxbcdt[:, :_D_INNER]
    xbc = zxbcdt[:, _D_INNER:_D_INNER + _CONV_DIM]
    dt_raw = zxbcdt[:, _D_INNER + _CONV_DIM:]
    dt = jax.nn.softplus(dt_raw + dtb_ref[0])

    convw = convw_ref[0]
    taps = []
    for nb in range(_BATCH):
        xbc_b = xbc[nb * _CHUNK:(nb + 1) * _CHUNK]
        xfull = jnp.concatenate(
            [carry_ref[nb * 8:(nb + 1) * 8], xbc_b], axis=0)
        carry_ref[nb * 8:(nb + 1) * 8] = xbc_b[_CHUNK - 8:_CHUNK]
        taps.append(xfull[5:5 + _CHUNK] * convw[0:1]
                    + xfull[6:6 + _CHUNK] * convw[1:2]
                    + xfull[7:7 + _CHUNK] * convw[2:3]
                    + xbc_b * convw[3:4])
    conv = jnp.concatenate(taps, axis=0) + convb_ref[0]
    xconv = conv * jax.nn.sigmoid(conv)

    xs = xconv[:, :_D_INNER]
    Bmat = xconv[:, _D_INNER:_D_INNER + _D_STATE]
    Cmat = xconv[:, _D_INNER + _D_STATE:]

    A2 = -jnp.exp(alog_ref[0]) * _LOG2E
    a_log2 = dt * A2
    row = jax.lax.broadcasted_iota(jnp.int32, (_R, _R), 0)
    col = jax.lax.broadcasted_iota(jnp.int32, (_R, _R), 1)
    trilbd = ((row >= col) & (row // _CHUNK == col // _CHUNK)
              ).astype(jnp.float32)
    cum2 = jnp.dot(trilbd, a_log2, preferred_element_type=jnp.float32)
    cum2_T = cum2.T

    e64 = e64_ref[...]
    cumB = jnp.dot(cum2, e256_ref[...],
                   preferred_element_type=jnp.float32)
    ecum = jnp.exp2(cum2)
    last0 = cum2[_CHUNK - 1:_CHUNK, :]
    last1 = cum2[_R - 1:_R, :]
    dec = jnp.exp2(jnp.concatenate(
        [last0 - cum2[:_CHUNK], last1 - cum2[_CHUNK:]], axis=0))
    stacked = jnp.concatenate([dt, ecum, dec], axis=0)
    bcast = jnp.dot(stacked, e64, preferred_element_type=jnp.float32)
    dtB = bcast[:_R]
    ecumB = bcast[_R:2 * _R]
    decB = bcast[2 * _R:]

    Xdt = xs * dtB
    Xdt_b = Xdt.astype(jnp.bfloat16)
    Xdec_b = (Xdt * decB).astype(jnp.bfloat16)

    tril = trilbd[:_CHUNK, :_CHUNK]
    ys = [None] * _BATCH
    for nb in range(_BATCH):
        r0 = nb * _CHUNK
        sl = slice(r0, r0 + _CHUNK)
        Bmat_n = Bmat[sl]
        Cmat_n = Cmat[sl]
        h_n = h_ref[nb * _D_STATE:(nb + 1) * _D_STATE]
        Yoff = jnp.dot(Cmat_n.astype(jnp.bfloat16),
                       h_n.astype(jnp.bfloat16),
                       preferred_element_type=jnp.float32) * ecumB[sl]
        hdec = ecumB[r0 + _CHUNK - 1:r0 + _CHUNK]
        h_ref[nb * _D_STATE:(nb + 1) * _D_STATE] = h_n * hdec + \
            jax.lax.dot_general(
                Bmat_n.astype(jnp.bfloat16), Xdec_b[sl],
                (((0,), (0,)), ((), ())),
                preferred_element_type=jnp.float32)
        CBm = (jnp.dot(Cmat_n, Bmat_n.T, preferred_element_type=jnp.float32)
               * tril).astype(jnp.bfloat16)
        yh = []
        for hh in range(_NHEADS):
            seg = (cumB[sl, _CHUNK * hh:_CHUNK * (hh + 1)]
                   - cum2_T[hh:hh + 1, sl])
            Lh = jnp.exp2(jnp.minimum(seg, 0.0).astype(jnp.bfloat16))
            yh.append(jnp.dot(
                CBm * Lh, Xdt_b[sl, _HEADDIM * hh:_HEADDIM * (hh + 1)],
                preferred_element_type=jnp.float32))
        ys[nb] = jnp.concatenate(yh, axis=1) + Yoff
    Y = jnp.concatenate(ys, axis=0) + xs * dbig_ref[0]

    g = Y * z * jax.nn.sigmoid(z)
    ms = jnp.mean(g * g, axis=1, keepdims=True)
    gn = g * jax.lax.rsqrt(ms + 1e-5) * normw_ref[0]
    res = jnp.dot(gn, w_out_ref[0],
                  preferred_element_type=jnp.float32)

    @pl.when(l < _DEPTH - 1)
    def _():
        act_ref[0, pl.ds(c * _CHUNK, _CHUNK), :] = res[:_CHUNK]
        act_ref[1, pl.ds(c * _CHUNK, _CHUNK), :] = res[_CHUNK:]

    out_ref[0, 0] = res[:_CHUNK]
    out_ref[0, 1] = res[_CHUNK:]


def _selector(block):
    lane = jnp.arange(_NHEADS * block, dtype=jnp.int32)[None, :]
    sub = jnp.arange(_NHEADS, dtype=jnp.int32)[:, None]
    return (lane // block == sub).astype(jnp.float32)


def kernel(x, params):
    st = lambda k: jnp.stack([p[k] for p in params])
    full = lambda shape: pl.BlockSpec(shape, lambda l, c: tuple(0 for _ in shape))
    lfull = lambda shape: pl.BlockSpec((1,) + shape,
                                       lambda l, c: (l,) + tuple(0 for _ in shape))
    out = pl.pallas_call(
        _stack_kernel,
        grid=(_DEPTH, _NCHUNKS),
        in_specs=[
            pl.BlockSpec((1, _BATCH, _CHUNK, _D_MODEL), lambda l, c: (0, 0, c, 0)),
            lfull((_D_MODEL, _D_IN_PROJ)),
            lfull((4, _CONV_DIM)),
            lfull((1, _CONV_DIM)),
            lfull((1, _NHEADS)),
            lfull((1, _NHEADS)),
            lfull((1, _D_INNER)),
            lfull((1, _D_INNER)),
            lfull((_D_INNER, _D_MODEL)),
            full((_NHEADS, _D_INNER)),
            full((_NHEADS, _NHEADS * _CHUNK)),
        ],
        out_specs=pl.BlockSpec((1, _BATCH, _CHUNK, _D_MODEL),
                               lambda l, c: (l, 0, c, 0)),
        out_shape=jax.ShapeDtypeStruct((_DEPTH, _BATCH, _SEQLEN, _D_MODEL),
                                       jnp.float32),
        scratch_shapes=[
            pltpu.VMEM((_BATCH, _SEQLEN, _D_MODEL), jnp.float32),
            pltpu.VMEM((_BATCH * 8, _CONV_DIM), jnp.float32),
            pltpu.VMEM((_BATCH * _D_STATE, _NHEADS * _HEADDIM), jnp.float32),
        ],
        compiler_params=pltpu.CompilerParams(
            dimension_semantics=("arbitrary", "arbitrary"),
            vmem_limit_bytes=56 * 1024 * 1024,
        ),
    )(
        x[None],
        st("in_proj"),
        jnp.stack([p["conv_w"].T for p in params]),
        st("conv_b").reshape(_DEPTH, 1, _CONV_DIM),
        st("dt_bias").reshape(_DEPTH, 1, _NHEADS),
        st("A_log").reshape(_DEPTH, 1, _NHEADS),
        jnp.stack([jnp.repeat(p["D"], _HEADDIM).reshape(1, _D_INNER)
                   for p in params]),
        st("norm_w").reshape(_DEPTH, 1, _D_INNER),
        st("out_proj"),
        _selector(_HEADDIM),
        _selector(_CHUNK),
    )
    return out[_DEPTH - 1]
